# baseline (speedup 1.0000x reference)
import functools

import jax
import jax.numpy as jnp
from jax import lax
from jax.experimental import pallas as pl
from jax.experimental.pallas import tpu as pltpu
from jax.experimental.pallas import tpu_sc as plsc

B, L = 4096, 200
VOCAB, EMB = 128 * 128, 32
N = B * L

NC, NS = 2, 16
NW = NC * NS
UNIT = 1024
NQ = B // UNIT
NUNITS = L * NQ
UNITS_PER_W = NUNITS // NW
IDX_ROWS = UNIT // 128


def _tc_fold_table(emb_table, W, b1):
    def body(emb_ref, w_ref, b_ref, out_ref):
        acc = jnp.dot(emb_ref[...], w_ref[2:2 + EMB, :],
                      preferred_element_type=jnp.float32)
        out_ref[...] = acc + b_ref[...]

    return pl.pallas_call(
        body,
        out_shape=jax.ShapeDtypeStruct((VOCAB, EMB), jnp.float32),
    )(emb_table, W, b1)


def _sc_gather(t2, ids4d):
    mesh = plsc.VectorSubcoreMesh(core_axis_name="c", subcore_axis_name="s")

    @functools.partial(
        pl.kernel,
        out_type=jax.ShapeDtypeStruct((N, EMB), jnp.float32),
        mesh=mesh,
        scratch_types=[
            pltpu.VMEM((IDX_ROWS, 128), jnp.int32),
            pltpu.VMEM((UNIT, EMB), jnp.float32),
            pltpu.VMEM((IDX_ROWS, 128), jnp.int32),
            pltpu.VMEM((UNIT, EMB), jnp.float32),
            pltpu.SemaphoreType.DMA,
            pltpu.SemaphoreType.DMA,
            pltpu.SemaphoreType.DMA,
            pltpu.SemaphoreType.DMA,
        ],
        compiler_params=pltpu.CompilerParams(use_tc_tiling_on_sc=False),
    )
    def k(t2_hbm, ids_hbm, out_hbm,
          idx_a, rows_a, idx_b, rows_b, sga, sgb, swa, swb):
        wid = lax.axis_index("s") * NC + lax.axis_index("c")
        u0 = wid * UNITS_PER_W

        def start_unit(uu, idx_v, rows_v, sem):
            u = u0 + uu
            l = u // NQ
            q = u % NQ
            pltpu.sync_copy(ids_hbm.at[l, q], idx_v)
            return [
                pltpu.async_copy(
                    t2_hbm.at[idx_v.at[j]],
                    rows_v.at[pl.ds(j * 128, 128)],
                    sem,
                )
                for j in range(IDX_ROWS)
            ]

        def write_unit(uu, rows_v, sem):
            u = u0 + uu
            dst = pl.multiple_of(u * UNIT, UNIT)
            return pltpu.async_copy(rows_v, out_hbm.at[pl.ds(dst, UNIT), :],
                                    sem)

        bufs = ((idx_a, rows_a, sga, swa), (idx_b, rows_b, sgb, swb))
        pend_gather = [None, None]
        pend_write = [None, None]
        for uu in range(UNITS_PER_W + 1):
            pb = uu % 2
            if uu < UNITS_PER_W:
                idx_v, rows_v, sg, sw = bufs[pb]
                if pend_write[pb] is not None:
                    pend_write[pb].wait()
                    pend_write[pb] = None
                pend_gather[pb] = start_unit(uu, idx_v, rows_v, sg)
            ob = 1 - pb
            if pend_gather[ob] is not None:
                for cp in pend_gather[ob]:
                    cp.wait()
                pend_gather[ob] = None
                pend_write[ob] = write_unit(uu - 1, bufs[ob][1], bufs[ob][3])
        for pb in range(2):
            if pend_write[pb] is not None:
                pend_write[pb].wait()

    return k(t2, ids4d)


def _tc_affine_tanh(g, lng_t, lat_t, wc):
    def body(g_ref, lng_ref, lat_ref, w_ref, out_ref):
        gp = g_ref[...]
        g3 = gp.reshape(B // 4, 4, EMB)
        gt = jnp.transpose(g3, (2, 0, 1)).reshape(EMB, B)
        lng = lng_ref[0, 0, :][None, :]
        lat = lat_ref[0, 0, :][None, :]
        w0 = w_ref[0, :][:, None]
        w1 = w_ref[1, :][:, None]
        out_ref[0] = jnp.tanh(gt + lng * w0 + lat * w1)

    return pl.pallas_call(
        body,
        grid=(L,),
        in_specs=[
            pl.BlockSpec((B // 4, 128), lambda l: (l, 0)),
            pl.BlockSpec((1, 1, B), lambda l: (l, 0, 0)),
            pl.BlockSpec((1, 1, B), lambda l: (l, 0, 0)),
            pl.BlockSpec((2, EMB), lambda l: (0, 0)),
        ],
        out_specs=pl.BlockSpec((1, EMB, B), lambda l: (l, 0, 0)),
        out_shape=jax.ShapeDtypeStruct((L, EMB, B), jnp.float32),
    )(g, lng_t, lat_t, wc)


def kernel(lngs, lats, grid_id, emb_table, W, b):
    t2 = _tc_fold_table(emb_table, W, b.reshape(1, EMB))
    ids4d = grid_id.astype(jnp.int32).T.reshape(L, NQ, IDX_ROWS, 128)
    gathered = _sc_gather(t2, ids4d)
    g_packed = gathered.reshape(N * EMB // 128, 128)
    lng_t = lngs.T.reshape(L, 1, B)
    lat_t = lats.T.reshape(L, 1, B)
    out_t = _tc_affine_tanh(g_packed, lng_t, lat_t, W[0:2, :])
    return out_t.transpose(2, 0, 1)


# speedup vs baseline: 6.5430x; 6.5430x over previous
import functools

import jax
import jax.numpy as jnp
from jax import lax
from jax.experimental import pallas as pl
from jax.experimental.pallas import tpu as pltpu
from jax.experimental.pallas import tpu_sc as plsc

B, L = 4096, 200
VOCAB, EMB = 128 * 128, 32
N = B * L

NC, NS = 2, 16
NW = NC * NS
UNIT = 1024
NQ = B // UNIT
NUNITS = L * NQ
UNITS_PER_W = NUNITS // NW
IDX_ROWS = UNIT // 128


def _tc_fold_table(emb_table, W, b1):
    def body(emb_ref, w_ref, b_ref, out_ref):
        acc = jnp.dot(emb_ref[...], w_ref[2:2 + EMB, :],
                      preferred_element_type=jnp.float32)
        out_ref[...] = acc + b_ref[...]

    return pl.pallas_call(
        body,
        out_shape=jax.ShapeDtypeStruct((VOCAB, EMB), jnp.float32),
    )(emb_table, W, b1)


def _sc_gather(t2, ids4d):
    mesh = plsc.VectorSubcoreMesh(core_axis_name="c", subcore_axis_name="s")

    @functools.partial(
        pl.kernel,
        out_type=jax.ShapeDtypeStruct((N, EMB), jnp.float32),
        mesh=mesh,
        scratch_types=[
            pltpu.VMEM((IDX_ROWS, 128), jnp.int32),
            pltpu.VMEM((UNIT, EMB), jnp.float32),
            pltpu.VMEM((IDX_ROWS, 128), jnp.int32),
            pltpu.VMEM((UNIT, EMB), jnp.float32),
            pltpu.SemaphoreType.DMA,
            pltpu.SemaphoreType.DMA,
            pltpu.SemaphoreType.DMA,
            pltpu.SemaphoreType.DMA,
        ],
        compiler_params=pltpu.CompilerParams(use_tc_tiling_on_sc=False),
    )
    def k(t2_hbm, ids_hbm, out_hbm,
          idx_a, rows_a, idx_b, rows_b, sga, sgb, swa, swb):
        wid = lax.axis_index("s") * NC + lax.axis_index("c")
        u0 = wid * UNITS_PER_W

        def start_unit(uu, idx_v, rows_v, sem):
            u = u0 + uu
            l = u // NQ
            q = u % NQ
            pltpu.sync_copy(ids_hbm.at[l, q], idx_v)
            return [
                pltpu.async_copy(
                    t2_hbm.at[idx_v.at[j]],
                    rows_v.at[pl.ds(j * 128, 128)],
                    sem,
                )
                for j in range(IDX_ROWS)
            ]

        def write_unit(uu, rows_v, sem):
            u = u0 + uu
            dst = pl.multiple_of(u * UNIT, UNIT)
            return pltpu.async_copy(rows_v, out_hbm.at[pl.ds(dst, UNIT), :],
                                    sem)

        bufs = ((idx_a, rows_a, sga, swa), (idx_b, rows_b, sgb, swb))
        pend_gather = [None, None]
        pend_write = [None, None]
        for uu in range(UNITS_PER_W + 1):
            pb = uu % 2
            if uu < UNITS_PER_W:
                idx_v, rows_v, sg, sw = bufs[pb]
                if pend_write[pb] is not None:
                    pend_write[pb].wait()
                    pend_write[pb] = None
                pend_gather[pb] = start_unit(uu, idx_v, rows_v, sg)
            ob = 1 - pb
            if pend_gather[ob] is not None:
                for cp in pend_gather[ob]:
                    cp.wait()
                pend_gather[ob] = None
                pend_write[ob] = write_unit(uu - 1, bufs[ob][1], bufs[ob][3])
        for pb in range(2):
            if pend_write[pb] is not None:
                pend_write[pb].wait()

    return k(t2, ids4d)


def _tc_affine_tanh(g, lng_t, lat_t, wc):
    def body(g_ref, lng_ref, lat_ref, w_ref, out_ref):
        w0 = w_ref[0, :][:, None]
        w1 = w_ref[1, :][:, None]
        for tr in range(8):
            tt = g_ref[pl.ds(tr * 128, 128), :].T
            for kk in range(4):
                bk = tr * 4 + kk
                lng_sl = lng_ref[0, 0, pl.ds(bk * 128, 128)][None, :]
                lat_sl = lat_ref[0, 0, pl.ds(bk * 128, 128)][None, :]
                y = tt[kk * 32:(kk + 1) * 32, :] + lng_sl * w0 + lat_sl * w1
                out_ref[0, :, bk] = jnp.tanh(y).reshape(EMB // 8, 8, 128)

    return pl.pallas_call(
        body,
        grid=(L,),
        in_specs=[
            pl.BlockSpec((B // 4, 128), lambda l: (l, 0)),
            pl.BlockSpec((1, 1, B), lambda l: (l, 0, 0)),
            pl.BlockSpec((1, 1, B), lambda l: (l, 0, 0)),
            pl.BlockSpec((2, EMB), lambda l: (0, 0)),
        ],
        out_specs=pl.BlockSpec((1, EMB // 8, B // 128, 8, 128),
                               lambda l: (l, 0, 0, 0, 0)),
        out_shape=jax.ShapeDtypeStruct((L, EMB // 8, B // 128, 8, 128),
                                       jnp.float32),
    )(g, lng_t, lat_t, wc)


def kernel(lngs, lats, grid_id, emb_table, W, b):
    t2 = _tc_fold_table(emb_table, W, b.reshape(1, EMB))
    ids4d = (grid_id.astype(jnp.int32).T
             .reshape(L, NQ, 2, 4, 128)
             .transpose(0, 1, 2, 4, 3)
             .reshape(L, NQ, IDX_ROWS, 128))
    gathered = _sc_gather(t2, ids4d)
    g_packed = gathered.reshape(N * EMB // 128, 128)
    lng_t = lngs.T.reshape(L, 1, B)
    lat_t = lats.T.reshape(L, 1, B)
    out5 = _tc_affine_tanh(g_packed, lng_t, lat_t, W[0:2, :])
    return out5.transpose(2, 4, 0, 1, 3).reshape(B, L, EMB)


# speedup vs baseline: 7.0207x; 1.0730x over previous
import functools

import jax
import jax.numpy as jnp
from jax import lax
from jax.experimental import pallas as pl
from jax.experimental.pallas import tpu as pltpu
from jax.experimental.pallas import tpu_sc as plsc

B, L = 4096, 200
VOCAB, EMB = 128 * 128, 32
N = B * L

NC, NS = 2, 16
NW = NC * NS
UNIT = 1024
NQ = B // UNIT
IDX_ROWS = UNIT // 128
NSLICE = 5
LSL = L // NSLICE
UNITS_PER_W = LSL * NQ // NW


def _tc_fold_table(emb_table, W, b1):
    def body(emb_ref, w_ref, b_ref, out_ref):
        acc = jnp.dot(emb_ref[...], w_ref[2:2 + EMB, :],
                      preferred_element_type=jnp.float32)
        out_ref[...] = acc + b_ref[...]

    return pl.pallas_call(
        body,
        out_shape=jax.ShapeDtypeStruct((VOCAB, EMB), jnp.float32),
    )(emb_table, W, b1)


def _sc_gather_slice(t2, ids4d, l0):
    mesh = plsc.VectorSubcoreMesh(core_axis_name="c", subcore_axis_name="s")

    @functools.partial(
        pl.kernel,
        out_type=jax.ShapeDtypeStruct((LSL * B, EMB), jnp.float32),
        mesh=mesh,
        scratch_types=[
            pltpu.VMEM((IDX_ROWS, 128), jnp.int32),
            pltpu.VMEM((UNIT, EMB), jnp.float32),
            pltpu.VMEM((IDX_ROWS, 128), jnp.int32),
            pltpu.VMEM((UNIT, EMB), jnp.float32),
            pltpu.SemaphoreType.DMA,
            pltpu.SemaphoreType.DMA,
            pltpu.SemaphoreType.DMA,
            pltpu.SemaphoreType.DMA,
        ],
        compiler_params=pltpu.CompilerParams(use_tc_tiling_on_sc=False),
    )
    def k(t2_hbm, ids_hbm, out_hbm,
          idx_a, rows_a, idx_b, rows_b, sga, sgb, swa, swb):
        wid = lax.axis_index("s") * NC + lax.axis_index("c")
        u0 = NQ * l0 + wid * UNITS_PER_W

        def start_unit(uu, idx_v, rows_v, sem):
            u = u0 + uu
            l = u // NQ
            q = u % NQ
            pltpu.sync_copy(ids_hbm.at[l, q], idx_v)
            return [
                pltpu.async_copy(
                    t2_hbm.at[idx_v.at[j]],
                    rows_v.at[pl.ds(j * 128, 128)],
                    sem,
                )
                for j in range(IDX_ROWS)
            ]

        def write_unit(uu, rows_v, sem):
            u = u0 + uu
            dst = pl.multiple_of((u - NQ * l0) * UNIT, UNIT)
            return pltpu.async_copy(rows_v, out_hbm.at[pl.ds(dst, UNIT), :],
                                    sem)

        bufs = ((idx_a, rows_a, sga, swa), (idx_b, rows_b, sgb, swb))
        pend_gather = [None, None]
        pend_write = [None, None]
        for uu in range(UNITS_PER_W + 1):
            pb = uu % 2
            if uu < UNITS_PER_W:
                idx_v, rows_v, sg, sw = bufs[pb]
                if pend_write[pb] is not None:
                    pend_write[pb].wait()
                    pend_write[pb] = None
                pend_gather[pb] = start_unit(uu, idx_v, rows_v, sg)
            ob = 1 - pb
            if pend_gather[ob] is not None:
                for cp in pend_gather[ob]:
                    cp.wait()
                pend_gather[ob] = None
                pend_write[ob] = write_unit(uu - 1, bufs[ob][1], bufs[ob][3])
        for pb in range(2):
            if pend_write[pb] is not None:
                pend_write[pb].wait()

    return k(t2, ids4d)


def _epilogue_body(g_ref, lng_ref, lat_ref, w_ref, out_ref):
    w0 = w_ref[0, :][:, None]
    w1 = w_ref[1, :][:, None]
    for tr in range(8):
        tt = g_ref[pl.ds(tr * 128, 128), :].T
        for kk in range(4):
            bk = tr * 4 + kk
            lng_sl = lng_ref[0, 0, pl.ds(bk * 128, 128)][None, :]
            lat_sl = lat_ref[0, 0, pl.ds(bk * 128, 128)][None, :]
            y = tt[kk * 32:(kk + 1) * 32, :] + lng_sl * w0 + lat_sl * w1
            out_ref[0, :, bk] = jnp.tanh(y).reshape(EMB // 8, 8, 128)


def _tc_affine_tanh_slice(g, lng_t, lat_t, wc, acc, l0):
    args = (g, lng_t, lat_t, wc)
    in_specs = [
        pl.BlockSpec((B // 4, 128), lambda l: (l, 0)),
        pl.BlockSpec((1, 1, B), lambda l: (l0 + l, 0, 0)),
        pl.BlockSpec((1, 1, B), lambda l: (l0 + l, 0, 0)),
        pl.BlockSpec((2, EMB), lambda l: (0, 0)),
    ]
    if acc is not None:
        def body(g_ref, lng_ref, lat_ref, w_ref, acc_ref, out_ref):
            del acc_ref
            _epilogue_body(g_ref, lng_ref, lat_ref, w_ref, out_ref)

        args = args + (acc,)
        in_specs = in_specs + [
            pl.BlockSpec(memory_space=pltpu.MemorySpace.HBM)]
        io_aliases = {4: 0}
    else:
        body = _epilogue_body
        io_aliases = {}

    return pl.pallas_call(
        body,
        grid=(LSL,),
        in_specs=in_specs,
        out_specs=pl.BlockSpec((1, EMB // 8, B // 128, 8, 128),
                               lambda l: (l0 + l, 0, 0, 0, 0)),
        out_shape=jax.ShapeDtypeStruct((L, EMB // 8, B // 128, 8, 128),
                                       jnp.float32),
        input_output_aliases=io_aliases,
    )(*args)


def kernel(lngs, lats, grid_id, emb_table, W, b):
    t2 = _tc_fold_table(emb_table, W, b.reshape(1, EMB))
    ids4d = (grid_id.astype(jnp.int32).T
             .reshape(L, NQ, 2, 4, 128)
             .transpose(0, 1, 2, 4, 3)
             .reshape(L, NQ, IDX_ROWS, 128))
    lng_t = lngs.T.reshape(L, 1, B)
    lat_t = lats.T.reshape(L, 1, B)
    wc = W[0:2, :]

    acc = None
    for i in range(NSLICE):
        l0 = i * LSL
        g = _sc_gather_slice(t2, ids4d, l0)
        g_packed = g.reshape(LSL * B * EMB // 128, 128)
        acc = _tc_affine_tanh_slice(g_packed, lng_t, lat_t, wc, acc, l0)
    return acc.transpose(2, 4, 0, 1, 3).reshape(B, L, EMB)


# speedup vs baseline: 8.3797x; 1.1936x over previous
import functools

import jax
import jax.numpy as jnp
from jax import lax
from jax.experimental import pallas as pl
from jax.experimental.pallas import tpu as pltpu
from jax.experimental.pallas import tpu_sc as plsc

B, L = 4096, 200
VOCAB, EMB = 128 * 128, 32
N = B * L

NC, NS = 2, 16
NW = NC * NS
UNIT = 1024
NQ = B // UNIT
IDX_ROWS = UNIT // 128
NSLICE = 5
LSL = L // NSLICE
UNITS_PER_W = LSL * NQ // NW


def _tc_fold_table(emb_table, W, b1):
    def body(emb_ref, w_ref, b_ref, out_ref):
        acc = jnp.dot(emb_ref[...], w_ref[2:2 + EMB, :],
                      preferred_element_type=jnp.float32)
        acc = acc + b_ref[...]
        lo = jax.lax.bitcast_convert_type(
            acc[:, :EMB // 2].astype(jnp.bfloat16), jnp.uint16)
        hi = jax.lax.bitcast_convert_type(
            acc[:, EMB // 2:].astype(jnp.bfloat16), jnp.uint16)
        word = lo.astype(jnp.uint32) | (hi.astype(jnp.uint32) << 16)
        out_ref[...] = jax.lax.bitcast_convert_type(word, jnp.float32)

    return pl.pallas_call(
        body,
        out_shape=jax.ShapeDtypeStruct((VOCAB, EMB // 2), jnp.float32),
    )(emb_table, W, b1)


def _sc_gather_slice(t2, ids4d, l0):
    mesh = plsc.VectorSubcoreMesh(core_axis_name="c", subcore_axis_name="s")

    @functools.partial(
        pl.kernel,
        out_type=jax.ShapeDtypeStruct((LSL * B, EMB // 2), jnp.float32),
        mesh=mesh,
        scratch_types=[
            pltpu.VMEM((IDX_ROWS, 128), jnp.int32),
            pltpu.VMEM((UNIT, EMB // 2), jnp.float32),
            pltpu.VMEM((IDX_ROWS, 128), jnp.int32),
            pltpu.VMEM((UNIT, EMB // 2), jnp.float32),
            pltpu.SemaphoreType.DMA,
            pltpu.SemaphoreType.DMA,
            pltpu.SemaphoreType.DMA,
            pltpu.SemaphoreType.DMA,
        ],
        compiler_params=pltpu.CompilerParams(use_tc_tiling_on_sc=False),
    )
    def k(t2_hbm, ids_hbm, out_hbm,
          idx_a, rows_a, idx_b, rows_b, sga, sgb, swa, swb):
        wid = lax.axis_index("s") * NC + lax.axis_index("c")
        u0 = NQ * l0 + wid * UNITS_PER_W

        def start_unit(uu, idx_v, rows_v, sem):
            u = u0 + uu
            l = u // NQ
            q = u % NQ
            pltpu.sync_copy(ids_hbm.at[l, q], idx_v)
            return [
                pltpu.async_copy(
                    t2_hbm.at[idx_v.at[j]],
                    rows_v.at[pl.ds(j * 128, 128)],
                    sem,
                )
                for j in range(IDX_ROWS)
            ]

        def write_unit(uu, rows_v, sem):
            u = u0 + uu
            dst = pl.multiple_of((u - NQ * l0) * UNIT, UNIT)
            return pltpu.async_copy(rows_v, out_hbm.at[pl.ds(dst, UNIT), :],
                                    sem)

        bufs = ((idx_a, rows_a, sga, swa), (idx_b, rows_b, sgb, swb))
        pend_gather = [None, None]
        pend_write = [None, None]
        for uu in range(UNITS_PER_W + 1):
            pb = uu % 2
            if uu < UNITS_PER_W:
                idx_v, rows_v, sg, sw = bufs[pb]
                if pend_write[pb] is not None:
                    pend_write[pb].wait()
                    pend_write[pb] = None
                pend_gather[pb] = start_unit(uu, idx_v, rows_v, sg)
            ob = 1 - pb
            if pend_gather[ob] is not None:
                for cp in pend_gather[ob]:
                    cp.wait()
                pend_gather[ob] = None
                pend_write[ob] = write_unit(uu - 1, bufs[ob][1], bufs[ob][3])
        for pb in range(2):
            if pend_write[pb] is not None:
                pend_write[pb].wait()

    return k(t2, ids4d)


def _epilogue_body(g_ref, lng_ref, lat_ref, w_ref, out_ref):
    w0lo = w_ref[0, :EMB // 2][:, None]
    w0hi = w_ref[0, EMB // 2:][:, None]
    w1lo = w_ref[1, :EMB // 2][:, None]
    w1hi = w_ref[1, EMB // 2:][:, None]
    for tr in range(4):
        tt = g_ref[pl.ds(tr * 128, 128), :].T
        tw = jax.lax.bitcast_convert_type(tt, jnp.uint32)
        for kk in range(8):
            bk = tr * 8 + kk
            words = tw[kk * 16:(kk + 1) * 16, :]
            glo = jax.lax.bitcast_convert_type(words << 16, jnp.float32)
            ghi = jax.lax.bitcast_convert_type(
                words & jnp.uint32(0xFFFF0000), jnp.float32)
            lng_sl = lng_ref[0, 0, pl.ds(bk * 128, 128)][None, :]
            lat_sl = lat_ref[0, 0, pl.ds(bk * 128, 128)][None, :]
            ylo = jnp.tanh(glo + lng_sl * w0lo + lat_sl * w1lo)
            yhi = jnp.tanh(ghi + lng_sl * w0hi + lat_sl * w1hi)
            y = jnp.concatenate([ylo, yhi], axis=0)
            out_ref[0, :, bk] = y.reshape(EMB // 8, 8, 128)


def _tc_affine_tanh_slice(g, lng_t, lat_t, wc, acc, l0):
    args = (g, lng_t, lat_t, wc)
    in_specs = [
        pl.BlockSpec((B // 8, 128), lambda l: (l, 0)),
        pl.BlockSpec((1, 1, B), lambda l: (l0 + l, 0, 0)),
        pl.BlockSpec((1, 1, B), lambda l: (l0 + l, 0, 0)),
        pl.BlockSpec((2, EMB), lambda l: (0, 0)),
    ]
    if acc is not None:
        def body(g_ref, lng_ref, lat_ref, w_ref, acc_ref, out_ref):
            del acc_ref
            _epilogue_body(g_ref, lng_ref, lat_ref, w_ref, out_ref)

        args = args + (acc,)
        in_specs = in_specs + [
            pl.BlockSpec(memory_space=pltpu.MemorySpace.HBM)]
        io_aliases = {4: 0}
    else:
        body = _epilogue_body
        io_aliases = {}

    return pl.pallas_call(
        body,
        grid=(LSL,),
        in_specs=in_specs,
        out_specs=pl.BlockSpec((1, EMB // 8, B // 128, 8, 128),
                               lambda l: (l0 + l, 0, 0, 0, 0)),
        out_shape=jax.ShapeDtypeStruct((L, EMB // 8, B // 128, 8, 128),
                                       jnp.float32),
        input_output_aliases=io_aliases,
    )(*args)


def kernel(lngs, lats, grid_id, emb_table, W, b):
    t2 = _tc_fold_table(emb_table, W, b.reshape(1, EMB))
    ids4d = (grid_id.astype(jnp.int32).T
             .reshape(L, NQ, IDX_ROWS, 128)
             .transpose(0, 1, 3, 2)
             .reshape(L, NQ, IDX_ROWS, 128))
    lng_t = lngs.T.reshape(L, 1, B)
    lat_t = lats.T.reshape(L, 1, B)
    wc = W[0:2, :]

    acc = None
    for i in range(NSLICE):
        l0 = i * LSL
        g = _sc_gather_slice(t2, ids4d, l0)
        g_packed = g.reshape(LSL * B * (EMB // 2) // 128, 128)
        acc = _tc_affine_tanh_slice(g_packed, lng_t, lat_t, wc, acc, l0)
    return acc.transpose(2, 4, 0, 1, 3).reshape(B, L, EMB)


# speedup vs baseline: 10.3930x; 1.2403x over previous
import functools

import jax
import jax.numpy as jnp
from jax import lax
from jax.experimental import pallas as pl
from jax.experimental.pallas import tpu as pltpu
from jax.experimental.pallas import tpu_sc as plsc

B, L = 4096, 200
VOCAB, EMB = 128 * 128, 32
N = B * L

NC, NS = 2, 16
NW = NC * NS
UNIT = 1024
NQ = B // UNIT
IDX_ROWS = UNIT // 128
NSLICE = 5
LSL = L // NSLICE
UNITS_PER_W = LSL * NQ // NW
LPB = 2


def _tc_fold_table(emb_table, W, b1):
    def body(emb_ref, w_ref, b_ref, out_ref):
        acc = jnp.dot(emb_ref[...], w_ref[2:2 + EMB, :],
                      preferred_element_type=jnp.float32)
        acc = acc + b_ref[...]
        lo = jax.lax.bitcast_convert_type(
            acc[:, :EMB // 2].astype(jnp.bfloat16), jnp.uint16)
        hi = jax.lax.bitcast_convert_type(
            acc[:, EMB // 2:].astype(jnp.bfloat16), jnp.uint16)
        word = lo.astype(jnp.uint32) | (hi.astype(jnp.uint32) << 16)
        out_ref[...] = jax.lax.bitcast_convert_type(word, jnp.float32)

    return pl.pallas_call(
        body,
        out_shape=jax.ShapeDtypeStruct((VOCAB, EMB // 2), jnp.float32),
    )(emb_table, W, b1)


def _sc_gather_slice(t2, ids4d, l0):
    mesh = plsc.VectorSubcoreMesh(core_axis_name="c", subcore_axis_name="s")

    @functools.partial(
        pl.kernel,
        out_type=jax.ShapeDtypeStruct((LSL * B, EMB // 2), jnp.float32),
        mesh=mesh,
        scratch_types=[
            pltpu.VMEM((IDX_ROWS, 128), jnp.int32),
            pltpu.VMEM((UNIT, EMB // 2), jnp.float32),
            pltpu.VMEM((IDX_ROWS, 128), jnp.int32),
            pltpu.VMEM((UNIT, EMB // 2), jnp.float32),
            pltpu.SemaphoreType.DMA,
            pltpu.SemaphoreType.DMA,
            pltpu.SemaphoreType.DMA,
            pltpu.SemaphoreType.DMA,
        ],
        compiler_params=pltpu.CompilerParams(use_tc_tiling_on_sc=False),
    )
    def k(t2_hbm, ids_hbm, out_hbm,
          idx_a, rows_a, idx_b, rows_b, sga, sgb, swa, swb):
        wid = lax.axis_index("s") * NC + lax.axis_index("c")
        u0 = NQ * l0 + wid * UNITS_PER_W

        def start_unit(uu, idx_v, rows_v, sem):
            u = u0 + uu
            l = u // NQ
            q = u % NQ
            pltpu.sync_copy(ids_hbm.at[l, q], idx_v)
            return [
                pltpu.async_copy(
                    t2_hbm.at[idx_v.at[j]],
                    rows_v.at[pl.ds(j * 128, 128)],
                    sem,
                )
                for j in range(IDX_ROWS)
            ]

        def write_unit(uu, rows_v, sem):
            u = u0 + uu
            dst = pl.multiple_of((u - NQ * l0) * UNIT, UNIT)
            return pltpu.async_copy(rows_v, out_hbm.at[pl.ds(dst, UNIT), :],
                                    sem)

        bufs = ((idx_a, rows_a, sga, swa), (idx_b, rows_b, sgb, swb))
        pend_gather = [None, None]
        pend_write = [None, None]
        for uu in range(UNITS_PER_W + 1):
            pb = uu % 2
            if uu < UNITS_PER_W:
                idx_v, rows_v, sg, sw = bufs[pb]
                if pend_write[pb] is not None:
                    pend_write[pb].wait()
                    pend_write[pb] = None
                pend_gather[pb] = start_unit(uu, idx_v, rows_v, sg)
            ob = 1 - pb
            if pend_gather[ob] is not None:
                for cp in pend_gather[ob]:
                    cp.wait()
                pend_gather[ob] = None
                pend_write[ob] = write_unit(uu - 1, bufs[ob][1], bufs[ob][3])
        for pb in range(2):
            if pend_write[pb] is not None:
                pend_write[pb].wait()

    return k(t2, ids4d)


def _epilogue_body(g_ref, lng_ref, lat_ref, w_ref, out_ref):
    w0lo = w_ref[0, :EMB // 2][:, None]
    w0hi = w_ref[0, EMB // 2:][:, None]
    w1lo = w_ref[1, :EMB // 2][:, None]
    w1hi = w_ref[1, EMB // 2:][:, None]
    for li in range(LPB):
        for tr in range(4):
            tt = g_ref[pl.ds(li * 512 + tr * 128, 128), :].T
            tw = jax.lax.bitcast_convert_type(tt, jnp.uint32)
            for kk in range(8):
                bk = tr * 8 + kk
                words = tw[kk * 16:(kk + 1) * 16, :]
                glo = jax.lax.bitcast_convert_type(words << 16, jnp.float32)
                ghi = jax.lax.bitcast_convert_type(
                    words & jnp.uint32(0xFFFF0000), jnp.float32)
                lng_sl = lng_ref[li, 0, pl.ds(bk * 128, 128)][None, :]
                lat_sl = lat_ref[li, 0, pl.ds(bk * 128, 128)][None, :]
                ylo = jnp.tanh(glo + lng_sl * w0lo + lat_sl * w1lo)
                yhi = jnp.tanh(ghi + lng_sl * w0hi + lat_sl * w1hi)
                y = jnp.concatenate([ylo, yhi], axis=0)
                out_ref[li, :, bk] = y.reshape(EMB // 8, 8, 128)


def _tc_affine_tanh_slice(g, lng_t, lat_t, wc, acc, l0):
    args = (g, lng_t, lat_t, wc)
    in_specs = [
        pl.BlockSpec((LPB * B // 8, 128), lambda l: (l, 0)),
        pl.BlockSpec((LPB, 1, B), lambda l: (l0 // LPB + l, 0, 0)),
        pl.BlockSpec((LPB, 1, B), lambda l: (l0 // LPB + l, 0, 0)),
        pl.BlockSpec((2, EMB), lambda l: (0, 0)),
    ]
    if acc is not None:
        def body(g_ref, lng_ref, lat_ref, w_ref, acc_ref, out_ref):
            del acc_ref
            _epilogue_body(g_ref, lng_ref, lat_ref, w_ref, out_ref)

        args = args + (acc,)
        in_specs = in_specs + [
            pl.BlockSpec(memory_space=pltpu.MemorySpace.HBM)]
        io_aliases = {4: 0}
    else:
        body = _epilogue_body
        io_aliases = {}

    return pl.pallas_call(
        body,
        grid=(LSL // LPB,),
        in_specs=in_specs,
        out_specs=pl.BlockSpec((LPB, EMB // 8, B // 128, 8, 128),
                               lambda l: (l0 // LPB + l, 0, 0, 0, 0)),
        out_shape=jax.ShapeDtypeStruct((L, EMB // 8, B // 128, 8, 128),
                                       jnp.float32),
        input_output_aliases=io_aliases,
    )(*args)


def kernel(lngs, lats, grid_id, emb_table, W, b):
    t2 = _tc_fold_table(emb_table, W, b.reshape(1, EMB))
    ids4d = (grid_id.astype(jnp.int32).T
             .reshape(L, NQ, IDX_ROWS, 128)
             .transpose(0, 1, 3, 2)
             .reshape(L, NQ, IDX_ROWS, 128))
    lng_t = lngs.T.reshape(L, 1, B)
    lat_t = lats.T.reshape(L, 1, B)
    wc = W[0:2, :]

    acc = None
    for i in range(NSLICE):
        l0 = i * LSL
        g = _sc_gather_slice(t2, ids4d, l0)
        g_packed = g.reshape(LSL * B * (EMB // 2) // 128, 128)
        acc = _tc_affine_tanh_slice(g_packed, lng_t, lat_t, wc, acc, l0)
    return acc.transpose(2, 4, 0, 1, 3).reshape(B, L, EMB)


# speedup vs baseline: 11.1291x; 1.0708x over previous
import functools

import jax
import jax.numpy as jnp
from jax import lax
from jax.experimental import pallas as pl
from jax.experimental.pallas import tpu as pltpu
from jax.experimental.pallas import tpu_sc as plsc

B, L = 4096, 200
VOCAB, EMB = 128 * 128, 32
N = B * L

NC, NS = 2, 16
NW = NC * NS
UNIT = 1024
NQ = B // UNIT
IDX_ROWS = UNIT // 128
NSLICE = 5
LSL = L // NSLICE
UNITS_PER_W = LSL * NQ // NW
LPB = 4


def _tc_fold_table(emb_table, W, b1):
    def body(emb_ref, w_ref, b_ref, out_ref):
        acc = jnp.dot(emb_ref[...], w_ref[2:2 + EMB, :],
                      preferred_element_type=jnp.float32)
        acc = acc + b_ref[...]
        lo = jax.lax.bitcast_convert_type(
            acc[:, :EMB // 2].astype(jnp.bfloat16), jnp.uint16)
        hi = jax.lax.bitcast_convert_type(
            acc[:, EMB // 2:].astype(jnp.bfloat16), jnp.uint16)
        word = lo.astype(jnp.uint32) | (hi.astype(jnp.uint32) << 16)
        out_ref[...] = jax.lax.bitcast_convert_type(word, jnp.float32)

    return pl.pallas_call(
        body,
        out_shape=jax.ShapeDtypeStruct((VOCAB, EMB // 2), jnp.float32),
    )(emb_table, W, b1)


def _sc_gather_slice(t2, ids4d, l0):
    mesh = plsc.VectorSubcoreMesh(core_axis_name="c", subcore_axis_name="s")

    @functools.partial(
        pl.kernel,
        out_type=jax.ShapeDtypeStruct((LSL * B, EMB // 2), jnp.float32),
        mesh=mesh,
        scratch_types=[
            pltpu.VMEM((IDX_ROWS, 128), jnp.int32),
            pltpu.VMEM((UNIT, EMB // 2), jnp.float32),
            pltpu.VMEM((IDX_ROWS, 128), jnp.int32),
            pltpu.VMEM((UNIT, EMB // 2), jnp.float32),
            pltpu.SemaphoreType.DMA,
            pltpu.SemaphoreType.DMA,
            pltpu.SemaphoreType.DMA,
            pltpu.SemaphoreType.DMA,
        ],
        compiler_params=pltpu.CompilerParams(use_tc_tiling_on_sc=False),
    )
    def k(t2_hbm, ids_hbm, out_hbm,
          idx_a, rows_a, idx_b, rows_b, sga, sgb, swa, swb):
        wid = lax.axis_index("s") * NC + lax.axis_index("c")
        u0 = NQ * l0 + wid * UNITS_PER_W

        def start_unit(uu, idx_v, rows_v, sem):
            u = u0 + uu
            l = u // NQ
            q = u % NQ
            pltpu.sync_copy(ids_hbm.at[l, q], idx_v)
            return [
                pltpu.async_copy(
                    t2_hbm.at[idx_v.at[j]],
                    rows_v.at[pl.ds(j * 128, 128)],
                    sem,
                )
                for j in range(IDX_ROWS)
            ]

        def write_unit(uu, rows_v, sem):
            u = u0 + uu
            dst = pl.multiple_of((u - NQ * l0) * UNIT, UNIT)
            return pltpu.async_copy(rows_v, out_hbm.at[pl.ds(dst, UNIT), :],
                                    sem)

        bufs = ((idx_a, rows_a, sga, swa), (idx_b, rows_b, sgb, swb))
        pend_gather = [None, None]
        pend_write = [None, None]
        for uu in range(UNITS_PER_W + 1):
            pb = uu % 2
            if uu < UNITS_PER_W:
                idx_v, rows_v, sg, sw = bufs[pb]
                if pend_write[pb] is not None:
                    pend_write[pb].wait()
                    pend_write[pb] = None
                pend_gather[pb] = start_unit(uu, idx_v, rows_v, sg)
            ob = 1 - pb
            if pend_gather[ob] is not None:
                for cp in pend_gather[ob]:
                    cp.wait()
                pend_gather[ob] = None
                pend_write[ob] = write_unit(uu - 1, bufs[ob][1], bufs[ob][3])
        for pb in range(2):
            if pend_write[pb] is not None:
                pend_write[pb].wait()

    return k(t2, ids4d)


def _epilogue_body(g_ref, lng_ref, lat_ref, w_ref, out_ref):
    w0lo = w_ref[0, :EMB // 2][:, None]
    w0hi = w_ref[0, EMB // 2:][:, None]
    w1lo = w_ref[1, :EMB // 2][:, None]
    w1hi = w_ref[1, EMB // 2:][:, None]
    for li in range(LPB):
        for tr in range(4):
            tt = g_ref[pl.ds(li * 512 + tr * 128, 128), :].T
            tw = jax.lax.bitcast_convert_type(tt, jnp.uint32)
            for kk in range(8):
                bk = tr * 8 + kk
                words = tw[kk * 16:(kk + 1) * 16, :]
                glo = jax.lax.bitcast_convert_type(words << 16, jnp.float32)
                ghi = jax.lax.bitcast_convert_type(
                    words & jnp.uint32(0xFFFF0000), jnp.float32)
                lng_sl = lng_ref[li, 0, pl.ds(bk * 128, 128)][None, :]
                lat_sl = lat_ref[li, 0, pl.ds(bk * 128, 128)][None, :]
                ylo = jnp.tanh(glo + lng_sl * w0lo + lat_sl * w1lo)
                yhi = jnp.tanh(ghi + lng_sl * w0hi + lat_sl * w1hi)
                y = jnp.concatenate([ylo, yhi], axis=0)
                out_ref[li, :, bk] = y.reshape(EMB // 8, 8, 128)


def _tc_affine_tanh_slice(g, lng_t, lat_t, wc, acc, l0):
    args = (g, lng_t, lat_t, wc)
    in_specs = [
        pl.BlockSpec((LPB * B // 8, 128), lambda l: (l, 0)),
        pl.BlockSpec((LPB, 1, B), lambda l: (l0 // LPB + l, 0, 0)),
        pl.BlockSpec((LPB, 1, B), lambda l: (l0 // LPB + l, 0, 0)),
        pl.BlockSpec((2, EMB), lambda l: (0, 0)),
    ]
    if acc is not None:
        def body(g_ref, lng_ref, lat_ref, w_ref, acc_ref, out_ref):
            del acc_ref
            _epilogue_body(g_ref, lng_ref, lat_ref, w_ref, out_ref)

        args = args + (acc,)
        in_specs = in_specs + [
            pl.BlockSpec(memory_space=pltpu.MemorySpace.HBM)]
        io_aliases = {4: 0}
    else:
        body = _epilogue_body
        io_aliases = {}

    return pl.pallas_call(
        body,
        grid=(LSL // LPB,),
        in_specs=in_specs,
        out_specs=pl.BlockSpec((LPB, EMB // 8, B // 128, 8, 128),
                               lambda l: (l0 // LPB + l, 0, 0, 0, 0)),
        out_shape=jax.ShapeDtypeStruct((L, EMB // 8, B // 128, 8, 128),
                                       jnp.float32),
        input_output_aliases=io_aliases,
    )(*args)


def kernel(lngs, lats, grid_id, emb_table, W, b):
    t2 = _tc_fold_table(emb_table, W, b.reshape(1, EMB))
    ids4d = (grid_id.astype(jnp.int32).T
             .reshape(L, NQ, IDX_ROWS, 128)
             .transpose(0, 1, 3, 2)
             .reshape(L, NQ, IDX_ROWS, 128))
    lng_t = lngs.T.reshape(L, 1, B)
    lat_t = lats.T.reshape(L, 1, B)
    wc = W[0:2, :]

    acc = None
    for i in range(NSLICE):
        l0 = i * LSL
        g = _sc_gather_slice(t2, ids4d, l0)
        g_packed = g.reshape(LSL * B * (EMB // 2) // 128, 128)
        acc = _tc_affine_tanh_slice(g_packed, lng_t, lat_t, wc, acc, l0)
    return acc.transpose(2, 4, 0, 1, 3).reshape(B, L, EMB)


# speedup vs baseline: 11.4084x; 1.0251x over previous
import functools

import jax
import jax.numpy as jnp
from jax import lax
from jax.experimental import pallas as pl
from jax.experimental.pallas import tpu as pltpu
from jax.experimental.pallas import tpu_sc as plsc

B, L = 4096, 200
VOCAB, EMB = 128 * 128, 32
N = B * L

NC, NS = 2, 16
NW = NC * NS
UNIT = 1024
NQ = B // UNIT
IDX_ROWS = UNIT // 128
NSLICE = 5
LSL = L // NSLICE
UNITS_PER_W = LSL * NQ // NW
LPB = 8


def _tc_fold_table(emb_table, W, b1):
    def body(emb_ref, w_ref, b_ref, out_ref):
        acc = jnp.dot(emb_ref[...], w_ref[2:2 + EMB, :],
                      preferred_element_type=jnp.float32)
        acc = acc + b_ref[...]
        lo = jax.lax.bitcast_convert_type(
            acc[:, :EMB // 2].astype(jnp.bfloat16), jnp.uint16)
        hi = jax.lax.bitcast_convert_type(
            acc[:, EMB // 2:].astype(jnp.bfloat16), jnp.uint16)
        word = lo.astype(jnp.uint32) | (hi.astype(jnp.uint32) << 16)
        out_ref[...] = jax.lax.bitcast_convert_type(word, jnp.float32)

    return pl.pallas_call(
        body,
        out_shape=jax.ShapeDtypeStruct((VOCAB, EMB // 2), jnp.float32),
    )(emb_table, W, b1)


def _sc_gather_slice(t2, ids4d, l0):
    mesh = plsc.VectorSubcoreMesh(core_axis_name="c", subcore_axis_name="s")

    @functools.partial(
        pl.kernel,
        out_type=jax.ShapeDtypeStruct((LSL * B, EMB // 2), jnp.float32),
        mesh=mesh,
        scratch_types=[
            pltpu.VMEM((IDX_ROWS, 128), jnp.int32),
            pltpu.VMEM((UNIT, EMB // 2), jnp.float32),
            pltpu.VMEM((IDX_ROWS, 128), jnp.int32),
            pltpu.VMEM((UNIT, EMB // 2), jnp.float32),
            pltpu.SemaphoreType.DMA,
            pltpu.SemaphoreType.DMA,
            pltpu.SemaphoreType.DMA,
            pltpu.SemaphoreType.DMA,
        ],
        compiler_params=pltpu.CompilerParams(use_tc_tiling_on_sc=False),
    )
    def k(t2_hbm, ids_hbm, out_hbm,
          idx_a, rows_a, idx_b, rows_b, sga, sgb, swa, swb):
        wid = lax.axis_index("s") * NC + lax.axis_index("c")
        u0 = NQ * l0 + wid * UNITS_PER_W

        def start_unit(uu, idx_v, rows_v, sem):
            u = u0 + uu
            l = u // NQ
            q = u % NQ
            pltpu.sync_copy(ids_hbm.at[l, q], idx_v)
            return [
                pltpu.async_copy(
                    t2_hbm.at[idx_v.at[j]],
                    rows_v.at[pl.ds(j * 128, 128)],
                    sem,
                )
                for j in range(IDX_ROWS)
            ]

        def write_unit(uu, rows_v, sem):
            u = u0 + uu
            dst = pl.multiple_of((u - NQ * l0) * UNIT, UNIT)
            return pltpu.async_copy(rows_v, out_hbm.at[pl.ds(dst, UNIT), :],
                                    sem)

        bufs = ((idx_a, rows_a, sga, swa), (idx_b, rows_b, sgb, swb))
        pend_gather = [None, None]
        pend_write = [None, None]
        for uu in range(UNITS_PER_W + 1):
            pb = uu % 2
            if uu < UNITS_PER_W:
                idx_v, rows_v, sg, sw = bufs[pb]
                if pend_write[pb] is not None:
                    pend_write[pb].wait()
                    pend_write[pb] = None
                pend_gather[pb] = start_unit(uu, idx_v, rows_v, sg)
            ob = 1 - pb
            if pend_gather[ob] is not None:
                for cp in pend_gather[ob]:
                    cp.wait()
                pend_gather[ob] = None
                pend_write[ob] = write_unit(uu - 1, bufs[ob][1], bufs[ob][3])
        for pb in range(2):
            if pend_write[pb] is not None:
                pend_write[pb].wait()

    return k(t2, ids4d)


def _epilogue_body(g_ref, lng_ref, lat_ref, w_ref, out_ref):
    w0lo = w_ref[0, :EMB // 2][:, None]
    w0hi = w_ref[0, EMB // 2:][:, None]
    w1lo = w_ref[1, :EMB // 2][:, None]
    w1hi = w_ref[1, EMB // 2:][:, None]
    for li in range(LPB):
        for tr in range(4):
            tt = g_ref[pl.ds(li * 512 + tr * 128, 128), :].T
            tw = jax.lax.bitcast_convert_type(tt, jnp.uint32)
            for kk in range(8):
                bk = tr * 8 + kk
                words = tw[kk * 16:(kk + 1) * 16, :]
                glo = jax.lax.bitcast_convert_type(words << 16, jnp.float32)
                ghi = jax.lax.bitcast_convert_type(
                    words & jnp.uint32(0xFFFF0000), jnp.float32)
                lng_sl = lng_ref[li, 0, pl.ds(bk * 128, 128)][None, :]
                lat_sl = lat_ref[li, 0, pl.ds(bk * 128, 128)][None, :]
                ylo = jnp.tanh(glo + lng_sl * w0lo + lat_sl * w1lo)
                yhi = jnp.tanh(ghi + lng_sl * w0hi + lat_sl * w1hi)
                y = jnp.concatenate([ylo, yhi], axis=0)
                out_ref[li, :, bk] = y.reshape(EMB // 8, 8, 128)


def _tc_affine_tanh_slice(g, lng_t, lat_t, wc, acc, l0):
    args = (g, lng_t, lat_t, wc)
    in_specs = [
        pl.BlockSpec((LPB * B // 8, 128), lambda l: (l, 0)),
        pl.BlockSpec((LPB, 1, B), lambda l: (l0 // LPB + l, 0, 0)),
        pl.BlockSpec((LPB, 1, B), lambda l: (l0 // LPB + l, 0, 0)),
        pl.BlockSpec((2, EMB), lambda l: (0, 0)),
    ]
    if acc is not None:
        def body(g_ref, lng_ref, lat_ref, w_ref, acc_ref, out_ref):
            del acc_ref
            _epilogue_body(g_ref, lng_ref, lat_ref, w_ref, out_ref)

        args = args + (acc,)
        in_specs = in_specs + [
            pl.BlockSpec(memory_space=pltpu.MemorySpace.HBM)]
        io_aliases = {4: 0}
    else:
        body = _epilogue_body
        io_aliases = {}

    return pl.pallas_call(
        body,
        grid=(LSL // LPB,),
        in_specs=in_specs,
        out_specs=pl.BlockSpec((LPB, EMB // 8, B // 128, 8, 128),
                               lambda l: (l0 // LPB + l, 0, 0, 0, 0)),
        out_shape=jax.ShapeDtypeStruct((L, EMB // 8, B // 128, 8, 128),
                                       jnp.float32),
        input_output_aliases=io_aliases,
    )(*args)


def kernel(lngs, lats, grid_id, emb_table, W, b):
    t2 = _tc_fold_table(emb_table, W, b.reshape(1, EMB))
    ids4d = (grid_id.astype(jnp.int32).T
             .reshape(L, NQ, IDX_ROWS, 128)
             .transpose(0, 1, 3, 2)
             .reshape(L, NQ, IDX_ROWS, 128))
    lng_t = lngs.T.reshape(L, 1, B)
    lat_t = lats.T.reshape(L, 1, B)
    wc = W[0:2, :]

    acc = None
    for i in range(NSLICE):
        l0 = i * LSL
        g = _sc_gather_slice(t2, ids4d, l0)
        g_packed = g.reshape(LSL * B * (EMB // 2) // 128, 128)
        acc = _tc_affine_tanh_slice(g_packed, lng_t, lat_t, wc, acc, l0)
    return acc.transpose(2, 4, 0, 1, 3).reshape(B, L, EMB)
